# baseline (device time: 24315 ns/iter reference)
import jax
import jax.numpy as jnp
from jax import lax
from jax.experimental import pallas as pl
from jax.experimental.pallas import tpu as pltpu

N_DEV = 4
N_EXP = 8
E_PER = 2
T_PER = 256
D_IN = 128
D_OUT = 256
CAPACITY = 102


def kernel(x, router_W, route_idx, expert_W):
    del router_W

    def body(x_ref, idx_ref, w_ref, out_ref,
             w_full, comm_w, comm_i,
             send_w, recv_w, send_i, recv_i):
        my = lax.axis_index("i")
        left = lax.rem(my + (N_DEV - 1), N_DEV)
        right = lax.rem(my + 1, N_DEV)

        barrier = pltpu.get_barrier_semaphore()
        for nbr in (left, right):
            pl.semaphore_signal(
                barrier, inc=1,
                device_id=(nbr,), device_id_type=pl.DeviceIdType.MESH,
            )
        pl.semaphore_wait(barrier, 2)

        exp_iota = lax.broadcasted_iota(jnp.int32, (T_PER, N_EXP), 1)
        onehot = (idx_ref[:, :] == exp_iota).astype(jnp.float32)

        w_full[pl.ds(my * E_PER, E_PER), :, :] = w_ref[:, :, :]
        comm_w[0, :, :, :] = w_ref[:, :, :]
        comm_i[0, :, :] = onehot

        base = jnp.zeros((N_EXP,), jnp.float32)

        for h in range(N_DEV - 1):
            ss = h % 2
            rs = (h + 1) % 2
            rdma_w = pltpu.make_async_remote_copy(
                src_ref=comm_w.at[ss], dst_ref=comm_w.at[rs],
                send_sem=send_w.at[ss], recv_sem=recv_w.at[rs],
                device_id=(right,), device_id_type=pl.DeviceIdType.MESH,
            )
            rdma_i = pltpu.make_async_remote_copy(
                src_ref=comm_i.at[ss], dst_ref=comm_i.at[rs],
                send_sem=send_i.at[ss], recv_sem=recv_i.at[rs],
                device_id=(right,), device_id_type=pl.DeviceIdType.MESH,
            )
            rdma_w.start()
            rdma_i.start()
            rdma_w.wait()
            rdma_i.wait()

            origin = lax.rem(my + (N_DEV - 1 - h), N_DEV)
            w_full[pl.ds(origin * E_PER, E_PER), :, :] = comm_w[rs, :, :, :]
            oh = comm_i[rs, :, :]
            base = base + jnp.where(
                origin < my, jnp.sum(oh, axis=0), jnp.zeros((N_EXP,), jnp.float32)
            )

        row = lax.broadcasted_iota(jnp.int32, (T_PER, T_PER), 0)
        col = lax.broadcasted_iota(jnp.int32, (T_PER, T_PER), 1)
        tri = (col <= row).astype(jnp.float32)
        prefix = jnp.dot(tri, onehot, preferred_element_type=jnp.float32)

        rank = base[None, :] + prefix
        rank_own = jnp.sum(rank * onehot, axis=1)
        keep = (rank_own <= float(CAPACITY)).astype(jnp.float32)
        gate = onehot * keep[:, None]

        acc = jnp.zeros((T_PER, D_OUT), jnp.float32)
        xv = x_ref[:, :]
        for e in range(N_EXP):
            acc = acc + jnp.dot(
                xv * gate[:, e:e + 1], w_full[e, :, :],
                preferred_element_type=jnp.float32,
            )
        out_ref[:, :] = acc

    return pl.pallas_call(
        body,
        out_shape=jax.ShapeDtypeStruct((T_PER, D_OUT), jnp.float32),
        in_specs=[
            pl.BlockSpec(memory_space=pltpu.VMEM),
            pl.BlockSpec(memory_space=pltpu.VMEM),
            pl.BlockSpec(memory_space=pltpu.VMEM),
        ],
        out_specs=pl.BlockSpec(memory_space=pltpu.VMEM),
        scratch_shapes=[
            pltpu.VMEM((N_EXP, D_IN, D_OUT), jnp.float32),
            pltpu.VMEM((2, E_PER, D_IN, D_OUT), jnp.float32),
            pltpu.VMEM((2, T_PER, N_EXP), jnp.float32),
            pltpu.SemaphoreType.DMA((2,)),
            pltpu.SemaphoreType.DMA((2,)),
            pltpu.SemaphoreType.DMA((2,)),
            pltpu.SemaphoreType.DMA((2,)),
        ],
        compiler_params=pltpu.CompilerParams(collective_id=0),
    )(x, route_idx, expert_W)


# device time: 14239 ns/iter; 1.7076x vs baseline; 1.7076x over previous
import jax
import jax.numpy as jnp
from jax import lax
from jax.experimental import pallas as pl
from jax.experimental.pallas import tpu as pltpu

N_DEV = 4
N_EXP = 8
E_PER = 2
T_PER = 256
D_IN = 128
D_OUT = 256
CAPACITY = 102


def kernel(x, router_W, route_idx, expert_W):
    del router_W

    def body(x_ref, idx_ref, w_ref, out_ref,
             comm_w, comm_c,
             send_w, recv_w, send_c, recv_c):
        my = lax.axis_index("i")

        barrier = pltpu.get_barrier_semaphore()
        for d in range(1, N_DEV):
            peer = lax.rem(my + d, N_DEV)
            pl.semaphore_signal(
                barrier, inc=1,
                device_id=(peer,), device_id_type=pl.DeviceIdType.MESH,
            )
        pl.semaphore_wait(barrier, N_DEV - 1)

        exp_iota = lax.broadcasted_iota(jnp.int32, (T_PER, N_EXP), 1)
        onehot = (idx_ref[:, :] == exp_iota).astype(jnp.float32)
        comm_c[my, :, :] = jnp.sum(onehot, axis=0, keepdims=True)
        comm_w[my, :, :, :] = w_ref[:, :, :]

        w_sends = []
        for d in range(1, N_DEV):
            peer = lax.rem(my + d, N_DEV)
            rw = pltpu.make_async_remote_copy(
                src_ref=comm_w.at[my], dst_ref=comm_w.at[my],
                send_sem=send_w.at[d - 1], recv_sem=recv_w.at[my],
                device_id=(peer,), device_id_type=pl.DeviceIdType.MESH,
            )
            rc = pltpu.make_async_remote_copy(
                src_ref=comm_c.at[my], dst_ref=comm_c.at[my],
                send_sem=send_c.at[d - 1], recv_sem=recv_c.at[my],
                device_id=(peer,), device_id_type=pl.DeviceIdType.MESH,
            )
            rw.start()
            rc.start()
            w_sends.append((rw, rc))

        xv = x_ref[:, :]

        def gate_col(e_id):
            return (idx_ref[:, :] == e_id).astype(jnp.float32)

        acc = jnp.dot(xv * gate_col(my * E_PER), w_ref[0, :, :],
                      preferred_element_type=jnp.float32)
        acc = acc + jnp.dot(xv * gate_col(my * E_PER + 1), w_ref[1, :, :],
                            preferred_element_type=jnp.float32)

        row = lax.broadcasted_iota(jnp.int32, (T_PER, T_PER), 0)
        col = lax.broadcasted_iota(jnp.int32, (T_PER, T_PER), 1)
        tri = (col <= row).astype(jnp.float32)
        prefix = jnp.dot(tri, onehot, preferred_element_type=jnp.float32)

        base = jnp.zeros((N_EXP,), jnp.float32)
        for d in range(1, N_DEV):
            peer = lax.rem(my + d, N_DEV)
            pltpu.make_async_remote_copy(
                src_ref=comm_c.at[peer], dst_ref=comm_c.at[peer],
                send_sem=send_c.at[d - 1], recv_sem=recv_c.at[peer],
                device_id=(peer,), device_id_type=pl.DeviceIdType.MESH,
            ).wait_recv()
            cnt = comm_c[peer, 0, :]
            base = base + jnp.where(peer < my, cnt,
                                    jnp.zeros((N_EXP,), jnp.float32))

        rank = base[None, :] + prefix
        rank_own = jnp.sum(rank * onehot, axis=1)
        keep = (rank_own <= float(CAPACITY)).astype(jnp.float32)

        for d in (1, 3, 2):
            peer = lax.rem(my + d, N_DEV)
            pltpu.make_async_remote_copy(
                src_ref=comm_w.at[peer], dst_ref=comm_w.at[peer],
                send_sem=send_w.at[d - 1], recv_sem=recv_w.at[peer],
                device_id=(peer,), device_id_type=pl.DeviceIdType.MESH,
            ).wait_recv()
            wp = comm_w[peer, :, :, :]
            acc = acc + jnp.dot(xv * gate_col(peer * E_PER), wp[0, :, :],
                                preferred_element_type=jnp.float32)
            acc = acc + jnp.dot(xv * gate_col(peer * E_PER + 1), wp[1, :, :],
                                preferred_element_type=jnp.float32)

        out_ref[:, :] = keep[:, None] * acc

        for rw, rc in w_sends:
            rw.wait_send()
            rc.wait_send()

    return pl.pallas_call(
        body,
        out_shape=jax.ShapeDtypeStruct((T_PER, D_OUT), jnp.float32),
        in_specs=[
            pl.BlockSpec(memory_space=pltpu.VMEM),
            pl.BlockSpec(memory_space=pltpu.VMEM),
            pl.BlockSpec(memory_space=pltpu.VMEM),
        ],
        out_specs=pl.BlockSpec(memory_space=pltpu.VMEM),
        scratch_shapes=[
            pltpu.VMEM((N_DEV, E_PER, D_IN, D_OUT), jnp.float32),
            pltpu.VMEM((N_DEV, 1, N_EXP), jnp.float32),
            pltpu.SemaphoreType.DMA((N_DEV - 1,)),
            pltpu.SemaphoreType.DMA((N_DEV,)),
            pltpu.SemaphoreType.DMA((N_DEV - 1,)),
            pltpu.SemaphoreType.DMA((N_DEV,)),
        ],
        compiler_params=pltpu.CompilerParams(collective_id=0),
    )(x, route_idx, expert_W)
